# baseline (speedup 1.0000x reference)
import functools

import jax
import jax.numpy as jnp
from jax.experimental import pallas as pl
from jax.experimental.pallas import tpu as pltpu
from jax.experimental.pallas import tpu_sc as plsc

N_ATOM = 50000
N_BOND = 50000
G = 512
BLK = 1000
NB = N_ATOM // BLK
IN_DIM = 128
GAT_HIDDEN = [32, 64, 128]
HEADS = 4
NEG_SLOPE = 0.2
N_ITERS = 5
N_LSTM_LAYERS = 3
NTYPES = ['atom', 'bond', 'global']
MECH = {'atom': [('bond', 'b2a'), ('global', 'g2a')],
        'bond': [('atom', 'a2b'), ('global', 'g2b')],
        'global': [('atom', 'a2g'), ('bond', 'b2g')]}

_ARB = pltpu.CompilerParams(dimension_semantics=("arbitrary",))


def _elu(x):
    return jnp.where(x > 0, x, jnp.exp(jnp.minimum(x, 0.0)) - 1.0)


def _lrelu(x):
    return jnp.where(x >= 0, x, NEG_SLOPE * x)


def _onehot(bcol_ref):
    b = bcol_ref[0]
    idx = jax.lax.broadcasted_iota(jnp.int32, (BLK, G), 1).astype(jnp.float32)
    return b == idx


def _seg_softmax(logits, seg, num):
    m = jax.ops.segment_max(logits, seg, num_segments=num)
    m = jnp.where(jnp.isfinite(m), m, 0.0)
    e = jnp.exp(logits - m[seg])
    s = jax.ops.segment_sum(e, seg, num_segments=num)
    return e / (s[seg] + 1e-9)


def _s2s_max_k(feat, qT, bcol, m_ref):
    i = pl.program_id(0)
    e = jnp.dot(feat[...], qT[...], preferred_element_type=jnp.float32)
    oh = _onehot(bcol)
    masked = jnp.where(oh, e, -jnp.inf)

    @pl.when(i == 0)
    def _():
        m_ref[...] = jnp.full((1, G), -jnp.inf, jnp.float32)

    m_ref[...] = jnp.maximum(m_ref[...], masked.max(axis=0, keepdims=True))


def _s2s_sum_k(feat, qT, bcol, msafe, u_ref, v_ref):
    i = pl.program_id(0)
    f = feat[...]
    e = jnp.dot(f, qT[...], preferred_element_type=jnp.float32)
    oh = _onehot(bcol)
    eh = jnp.exp(jnp.where(oh, e - msafe[...], -jnp.inf))

    @pl.when(i == 0)
    def _():
        u_ref[...] = jnp.zeros_like(u_ref)
        v_ref[...] = jnp.zeros_like(v_ref)

    u_ref[...] += eh.sum(axis=0, keepdims=True)
    v_ref[...] += jax.lax.dot_general(eh, f, (((0,), (0,)), ((), ())),
                                      preferred_element_type=jnp.float32)


def _s2s_attend(feat, qT, bcol):
    d = feat.shape[1]
    m = pl.pallas_call(
        _s2s_max_k,
        grid=(NB,),
        in_specs=[pl.BlockSpec((BLK, d), lambda i: (i, 0)),
                  pl.BlockSpec((d, G), lambda i: (0, 0)),
                  pl.BlockSpec((1, BLK, 1), lambda i: (i, 0, 0))],
        out_specs=pl.BlockSpec((1, G), lambda i: (0, 0)),
        out_shape=jax.ShapeDtypeStruct((1, G), jnp.float32),
        compiler_params=_ARB,
    )(feat, qT, bcol)
    msafe = jnp.where(jnp.isfinite(m), m, 0.0)
    u, v = pl.pallas_call(
        _s2s_sum_k,
        grid=(NB,),
        in_specs=[pl.BlockSpec((BLK, d), lambda i: (i, 0)),
                  pl.BlockSpec((d, G), lambda i: (0, 0)),
                  pl.BlockSpec((1, BLK, 1), lambda i: (i, 0, 0)),
                  pl.BlockSpec((1, G), lambda i: (0, 0))],
        out_specs=[pl.BlockSpec((1, G), lambda i: (0, 0)),
                   pl.BlockSpec((G, d), lambda i: (0, 0))],
        out_shape=[jax.ShapeDtypeStruct((1, G), jnp.float32),
                   jax.ShapeDtypeStruct((G, d), jnp.float32)],
        compiler_params=_ARB,
    )(feat, qT, bcol, msafe)
    return v / (u.reshape(G, 1) + 1e-9)


def _sel_mat(outf):
    hf = HEADS * outf
    row = jax.lax.broadcasted_iota(jnp.int32, (hf, HEADS), 0) // outf
    col = jax.lax.broadcasted_iota(jnp.int32, (hf, HEADS), 1)
    return (row == col).astype(jnp.float32)


def _glb_max_k(h, W, aflat, er, bcol, m_ref, *, outf):
    i = pl.program_id(0)
    hsrc = jnp.dot(h[...], W[...], preferred_element_type=jnp.float32)
    el = jnp.dot(hsrc * aflat[...], _sel_mat(outf),
                 preferred_element_type=jnp.float32)
    oh = _onehot(bcol)
    ohf = oh.astype(jnp.float32)
    erg = jnp.dot(ohf, er[...], preferred_element_type=jnp.float32)
    lg = _lrelu(el + erg)

    @pl.when(i == 0)
    def _():
        m_ref[...] = jnp.full((HEADS, G), -jnp.inf, jnp.float32)

    for hd in range(HEADS):
        mh = jnp.where(oh, lg[:, hd:hd + 1], -jnp.inf).max(axis=0, keepdims=True)
        m_ref[hd:hd + 1, :] = jnp.maximum(m_ref[hd:hd + 1, :], mh)


def _glb_sum_k(h, W, aflat, er, bcol, msafeT, u_ref, v_ref, *, outf):
    i = pl.program_id(0)
    hsrc = jnp.dot(h[...], W[...], preferred_element_type=jnp.float32)
    el = jnp.dot(hsrc * aflat[...], _sel_mat(outf),
                 preferred_element_type=jnp.float32)
    oh = _onehot(bcol)
    ohf = oh.astype(jnp.float32)
    erg = jnp.dot(ohf, er[...], preferred_element_type=jnp.float32)
    lg = _lrelu(el + erg)
    mg = jnp.dot(ohf, msafeT[...], preferred_element_type=jnp.float32)
    e = jnp.exp(lg - mg)

    @pl.when(i == 0)
    def _():
        u_ref[...] = jnp.zeros_like(u_ref)
        v_ref[...] = jnp.zeros_like(v_ref)

    u_ref[...] += jax.lax.dot_general(ohf, e, (((0,), (0,)), ((), ())),
                                      preferred_element_type=jnp.float32)
    for hd in range(HEADS):
        ow = ohf * e[:, hd:hd + 1]
        v_ref[:, hd * outf:(hd + 1) * outf] += jax.lax.dot_general(
            ow, hsrc[:, hd * outf:(hd + 1) * outf], (((0,), (0,)), ((), ())),
            preferred_element_type=jnp.float32)


def _glb_aggregate(h_src_nodes, W, a_src, er, bcol, outf):
    import functools
    fin = h_src_nodes.shape[1]
    hf = HEADS * outf
    aflat = a_src.reshape(1, hf)
    m = pl.pallas_call(
        functools.partial(_glb_max_k, outf=outf),
        grid=(NB,),
        in_specs=[pl.BlockSpec((BLK, fin), lambda i: (i, 0)),
                  pl.BlockSpec((fin, hf), lambda i: (0, 0)),
                  pl.BlockSpec((1, hf), lambda i: (0, 0)),
                  pl.BlockSpec((G, HEADS), lambda i: (0, 0)),
                  pl.BlockSpec((1, BLK, 1), lambda i: (i, 0, 0))],
        out_specs=pl.BlockSpec((HEADS, G), lambda i: (0, 0)),
        out_shape=jax.ShapeDtypeStruct((HEADS, G), jnp.float32),
        compiler_params=_ARB,
    )(h_src_nodes, W, aflat, er, bcol)
    msafeT = jnp.where(jnp.isfinite(m), m, 0.0).T
    u, v = pl.pallas_call(
        functools.partial(_glb_sum_k, outf=outf),
        grid=(NB,),
        in_specs=[pl.BlockSpec((BLK, fin), lambda i: (i, 0)),
                  pl.BlockSpec((fin, hf), lambda i: (0, 0)),
                  pl.BlockSpec((1, hf), lambda i: (0, 0)),
                  pl.BlockSpec((G, HEADS), lambda i: (0, 0)),
                  pl.BlockSpec((1, BLK, 1), lambda i: (i, 0, 0)),
                  pl.BlockSpec((G, HEADS), lambda i: (0, 0))],
        out_specs=[pl.BlockSpec((G, HEADS), lambda i: (0, 0)),
                   pl.BlockSpec((G, hf), lambda i: (0, 0))],
        out_shape=[jax.ShapeDtypeStruct((G, HEADS), jnp.float32),
                   jax.ShapeDtypeStruct((G, hf), jnp.float32)],
        compiler_params=_ARB,
    )(h_src_nodes, W, aflat, er, bcol, msafeT)
    rst = v.reshape(G, HEADS, outf) / (u[..., None] + 1e-9)
    return rst.reshape(G, hf)


def _upd_k(rst, x, resW, hgt, bcol, o_ref):
    oh = _onehot(bcol).astype(jnp.float32)
    out = rst[...] + jnp.dot(oh, hgt[...], preferred_element_type=jnp.float32)
    out += jnp.dot(x[...], resW[...], preferred_element_type=jnp.float32)
    o_ref[...] = _elu(out)


def _node_update(rst_edge, x, resW, hgt, bcol):
    fin = x.shape[1]
    hf = hgt.shape[1]
    return pl.pallas_call(
        _upd_k,
        grid=(NB,),
        in_specs=[pl.BlockSpec((BLK, hf), lambda i: (i, 0)),
                  pl.BlockSpec((BLK, fin), lambda i: (i, 0)),
                  pl.BlockSpec((fin, hf), lambda i: (0, 0)),
                  pl.BlockSpec((G, hf), lambda i: (0, 0)),
                  pl.BlockSpec((1, BLK, 1), lambda i: (i, 0, 0))],
        out_specs=pl.BlockSpec((BLK, hf), lambda i: (i, 0)),
        out_shape=jax.ShapeDtypeStruct((N_ATOM, hf), jnp.float32),
        compiler_params=_ARB,
    )(rst_edge, x, resW, hgt, bcol)


def _mm_k(x, W, o_ref):
    o_ref[...] = jnp.dot(x[...], W[...], preferred_element_type=jnp.float32)


def _dense_mm(x, W):
    fin, fout = W.shape
    return pl.pallas_call(
        _mm_k,
        grid=(NB,),
        in_specs=[pl.BlockSpec((BLK, fin), lambda i: (i, 0)),
                  pl.BlockSpec((fin, fout), lambda i: (0, 0))],
        out_specs=pl.BlockSpec((BLK, fout), lambda i: (i, 0)),
        out_shape=jax.ShapeDtypeStruct((x.shape[0], fout), jnp.float32),
        compiler_params=_ARB,
    )(x, W)


E_PAD = 102400


def _sc_gather(table, idx_p):
    D = table.shape[1]
    info = plsc.get_sparse_core_info()
    NW = info.num_cores * info.num_subcores
    b_per_w = E_PAD // NW
    C = 128
    n_chunks = b_per_w // C

    @functools.partial(
        pl.kernel,
        mesh=plsc.VectorSubcoreMesh(core_axis_name="c", subcore_axis_name="s"),
        out_type=jax.ShapeDtypeStruct((E_PAD, D), jnp.float32),
        scratch_types=[
            pltpu.VMEM((C,), jnp.int32),
            pltpu.VMEM((C, D), jnp.float32),
            pltpu.SemaphoreType.DMA,
        ],
    )
    def k(table_hbm, idx_hbm, out_hbm, idx_v, rows_v, sem):
        wid = jax.lax.axis_index("s") * info.num_cores + jax.lax.axis_index("c")
        base = wid * b_per_w

        def body(c, _):
            off = base + c * C
            pltpu.sync_copy(idx_hbm.at[pl.ds(off, C)], idx_v)
            pltpu.async_copy(table_hbm.at[idx_v], rows_v, sem).wait()
            pltpu.sync_copy(rows_v, out_hbm.at[pl.ds(off, C)])
            return 0

        jax.lax.fori_loop(0, n_chunks, body, 0)

    return k(table, idx_p)


def _gather_rows(table, idx):
    n = idx.shape[0]
    idx_p = jnp.concatenate(
        [idx.astype(jnp.int32), jnp.zeros((E_PAD - n,), jnp.int32)])
    return _sc_gather(table, idx_p)[:n]


def _fc_head_kernel(x_ref, w0, b0, w1, b1, w2, b2, w3, b3, o_ref):
    h = jnp.dot(x_ref[...], w0[...], preferred_element_type=jnp.float32) + b0[...]
    h = _elu(h)
    h = jnp.dot(h, w1[...], preferred_element_type=jnp.float32) + b1[...]
    h = _elu(h)
    h = jnp.dot(h, w2[...], preferred_element_type=jnp.float32) + b2[...]
    h = _elu(h)
    o_ref[...] = jnp.dot(h, w3[...], preferred_element_type=jnp.float32) + b3[...]


def _fc_head(x, fc):
    return pl.pallas_call(
        _fc_head_kernel,
        out_shape=jax.ShapeDtypeStruct((x.shape[0], 1), jnp.float32),
    )(x, fc[0]['W'], fc[0]['b'], fc[1]['W'], fc[1]['b'],
      fc[2]['W'], fc[2]['b'], fc[3]['W'], fc[3]['b'])


def kernel(feats_atom, feats_bond, feats_global, params, a2b_src, a2b_dst, atom_batch, bond_batch):
    atom_bcol = atom_batch.astype(jnp.float32).reshape(NB, BLK, 1)
    bond_bcol = bond_batch.astype(jnp.float32).reshape(NB, BLK, 1)
    bcols = {'atom': atom_bcol, 'bond': bond_bcol}
    nn_ = {'atom': N_ATOM, 'bond': N_BOND}
    h = {'atom': feats_atom, 'bond': feats_bond, 'global': feats_global}
    edges_ab = {'b2a': (a2b_dst, a2b_src), 'a2b': (a2b_src, a2b_dst)}

    for li, layer in enumerate(params['gat']):
        outf = GAT_HIDDEN[li]
        hf = HEADS * outf
        for m in NTYPES:
            pm = layer[m]
            x_m = h[m]
            if m == 'global':
                h_dst = (x_m @ pm['fc_master']).reshape(G, HEADS, outf)
                rst = jnp.zeros((G, hf), jnp.float32)
                for (s, e) in MECH[m]:
                    er = (h_dst * pm[e]['a_dst'][None]).sum(-1)
                    rst = rst + _glb_aggregate(h[s], pm[e]['W'], pm[e]['a_src'],
                                               er, bcols[s], outf)
                if li > 0:
                    rst = rst + x_m @ pm['res_fc']
                h[m] = jax.nn.elu(rst)
            else:
                Nm = nn_[m]
                (s, e) = MECH[m][0]
                src, dst = edges_ab[e]
                hsrc_flat = _dense_mm(h[s], pm[e]['W'])
                hdst_flat = _dense_mm(x_m, pm['fc_master'])
                el = (hsrc_flat.reshape(Nm, HEADS, outf)
                      * pm[e]['a_src'][None]).sum(-1)
                er = (hdst_flat.reshape(Nm, HEADS, outf)
                      * pm[e]['a_dst'][None]).sum(-1)
                logits = _lrelu(el[src] + er[dst])
                alpha = _seg_softmax(logits, dst, Nm)
                rows = _gather_rows(hsrc_flat, src).reshape(-1, HEADS, outf)
                rst_edge = jax.ops.segment_sum(rows * alpha[..., None],
                                               dst, num_segments=Nm)
                rst_edge = rst_edge.reshape(Nm, hf)
                (sg, eg) = MECH[m][1]
                hgt = (h['global'] @ pm[eg]['W']) / (1.0 + 1e-9)
                resW = pm['res_fc'] if li > 0 else jnp.zeros((x_m.shape[1], hf),
                                                             jnp.float32)
                h[m] = _node_update(rst_edge, x_m, resW, hgt, bcols[m])

    def lstm_forward(layers, x, hs, cs):
        inp = x
        nh = []
        nc = []
        for i, p in enumerate(layers):
            z = inp @ p['Wi'] + hs[i] @ p['Wh'] + p['b']
            ig, fg, gg, og = jnp.split(z, 4, axis=-1)
            c = jax.nn.sigmoid(fg) * cs[i] + jax.nn.sigmoid(ig) * jnp.tanh(gg)
            hcur = jax.nn.sigmoid(og) * jnp.tanh(c)
            nh.append(hcur)
            nc.append(c)
            inp = hcur
        return inp, nh, nc

    def set2set(feat, bcol, layers):
        d = feat.shape[1]
        q_star = jnp.zeros((G, 2 * d), jnp.float32)
        hs = [jnp.zeros((G, d), jnp.float32) for _ in range(N_LSTM_LAYERS)]
        cs = [jnp.zeros((G, d), jnp.float32) for _ in range(N_LSTM_LAYERS)]
        for _ in range(N_ITERS):
            q, hs, cs = lstm_forward(layers, q_star, hs, cs)
            r = _s2s_attend(feat, q.T, bcol)
            q_star = jnp.concatenate([q, r], axis=-1)
        return q_star

    r_atom = set2set(h['atom'], atom_bcol, params['set2set']['atom'])
    r_bond = set2set(h['bond'], bond_bcol, params['set2set']['bond'])
    out = jnp.concatenate([r_atom, r_bond, h['global']], axis=-1)
    return _fc_head(out, params['fc'])


# speedup vs baseline: 1.0387x; 1.0387x over previous
import functools

import jax
import jax.numpy as jnp
from jax.experimental import pallas as pl
from jax.experimental.pallas import tpu as pltpu
from jax.experimental.pallas import tpu_sc as plsc

N_ATOM = 50000
N_BOND = 50000
G = 512
BLK = 1000
NB = N_ATOM // BLK
IN_DIM = 128
GAT_HIDDEN = [32, 64, 128]
HEADS = 4
NEG_SLOPE = 0.2
N_ITERS = 5
N_LSTM_LAYERS = 3
NTYPES = ['atom', 'bond', 'global']
MECH = {'atom': [('bond', 'b2a'), ('global', 'g2a')],
        'bond': [('atom', 'a2b'), ('global', 'g2b')],
        'global': [('atom', 'a2g'), ('bond', 'b2g')]}

_ARB = pltpu.CompilerParams(dimension_semantics=("arbitrary",))


def _elu(x):
    return jnp.where(x > 0, x, jnp.exp(jnp.minimum(x, 0.0)) - 1.0)


def _lrelu(x):
    return jnp.where(x >= 0, x, NEG_SLOPE * x)


def _onehot(bcol_ref):
    b = bcol_ref[0]
    idx = jax.lax.broadcasted_iota(jnp.int32, (BLK, G), 1).astype(jnp.float32)
    return b == idx


def _seg_softmax(logits, seg, num):
    m = jax.ops.segment_max(logits, seg, num_segments=num)
    m = jnp.where(jnp.isfinite(m), m, 0.0)
    e = jnp.exp(logits - m[seg])
    s = jax.ops.segment_sum(e, seg, num_segments=num)
    return e / (s[seg] + 1e-9)


def _s2s_max_k(feat, qT, bcol, m_ref):
    i = pl.program_id(0)
    e = jnp.dot(feat[...], qT[...], preferred_element_type=jnp.float32)
    oh = _onehot(bcol)
    masked = jnp.where(oh, e, -jnp.inf)

    @pl.when(i == 0)
    def _():
        m_ref[...] = jnp.full((1, G), -jnp.inf, jnp.float32)

    m_ref[...] = jnp.maximum(m_ref[...], masked.max(axis=0, keepdims=True))


def _s2s_sum_k(feat, qT, bcol, msafe, u_ref, v_ref):
    i = pl.program_id(0)
    f = feat[...]
    e = jnp.dot(f, qT[...], preferred_element_type=jnp.float32)
    oh = _onehot(bcol)
    eh = jnp.exp(jnp.where(oh, e - msafe[...], -jnp.inf))

    @pl.when(i == 0)
    def _():
        u_ref[...] = jnp.zeros_like(u_ref)
        v_ref[...] = jnp.zeros_like(v_ref)

    u_ref[...] += eh.sum(axis=0, keepdims=True)
    v_ref[...] += jax.lax.dot_general(eh, f, (((0,), (0,)), ((), ())),
                                      preferred_element_type=jnp.float32)


def _s2s_attend(feat, qT, bcol):
    d = feat.shape[1]
    m = pl.pallas_call(
        _s2s_max_k,
        grid=(NB,),
        in_specs=[pl.BlockSpec((BLK, d), lambda i: (i, 0)),
                  pl.BlockSpec((d, G), lambda i: (0, 0)),
                  pl.BlockSpec((1, BLK, 1), lambda i: (i, 0, 0))],
        out_specs=pl.BlockSpec((1, G), lambda i: (0, 0)),
        out_shape=jax.ShapeDtypeStruct((1, G), jnp.float32),
        compiler_params=_ARB,
    )(feat, qT, bcol)
    msafe = jnp.where(jnp.isfinite(m), m, 0.0)
    u, v = pl.pallas_call(
        _s2s_sum_k,
        grid=(NB,),
        in_specs=[pl.BlockSpec((BLK, d), lambda i: (i, 0)),
                  pl.BlockSpec((d, G), lambda i: (0, 0)),
                  pl.BlockSpec((1, BLK, 1), lambda i: (i, 0, 0)),
                  pl.BlockSpec((1, G), lambda i: (0, 0))],
        out_specs=[pl.BlockSpec((1, G), lambda i: (0, 0)),
                   pl.BlockSpec((G, d), lambda i: (0, 0))],
        out_shape=[jax.ShapeDtypeStruct((1, G), jnp.float32),
                   jax.ShapeDtypeStruct((G, d), jnp.float32)],
        compiler_params=_ARB,
    )(feat, qT, bcol, msafe)
    return v / (u.reshape(G, 1) + 1e-9)


def _sel_mat(outf):
    hf = HEADS * outf
    row = jax.lax.broadcasted_iota(jnp.int32, (hf, HEADS), 0) // outf
    col = jax.lax.broadcasted_iota(jnp.int32, (hf, HEADS), 1)
    return (row == col).astype(jnp.float32)


def _glb_max_k(h, W, aflat, er, bcol, m_ref, *, outf):
    i = pl.program_id(0)
    hsrc = jnp.dot(h[...], W[...], preferred_element_type=jnp.float32)
    el = jnp.dot(hsrc * aflat[...], _sel_mat(outf),
                 preferred_element_type=jnp.float32)
    oh = _onehot(bcol)
    ohf = oh.astype(jnp.float32)
    erg = jnp.dot(ohf, er[...], preferred_element_type=jnp.float32)
    lg = _lrelu(el + erg)

    @pl.when(i == 0)
    def _():
        m_ref[...] = jnp.full((HEADS, G), -jnp.inf, jnp.float32)

    for hd in range(HEADS):
        mh = jnp.where(oh, lg[:, hd:hd + 1], -jnp.inf).max(axis=0, keepdims=True)
        m_ref[hd:hd + 1, :] = jnp.maximum(m_ref[hd:hd + 1, :], mh)


def _glb_sum_k(h, W, aflat, er, bcol, msafeT, u_ref, v_ref, *, outf):
    i = pl.program_id(0)
    hsrc = jnp.dot(h[...], W[...], preferred_element_type=jnp.float32)
    el = jnp.dot(hsrc * aflat[...], _sel_mat(outf),
                 preferred_element_type=jnp.float32)
    oh = _onehot(bcol)
    ohf = oh.astype(jnp.float32)
    erg = jnp.dot(ohf, er[...], preferred_element_type=jnp.float32)
    lg = _lrelu(el + erg)
    mg = jnp.dot(ohf, msafeT[...], preferred_element_type=jnp.float32)
    e = jnp.exp(lg - mg)

    @pl.when(i == 0)
    def _():
        u_ref[...] = jnp.zeros_like(u_ref)
        v_ref[...] = jnp.zeros_like(v_ref)

    u_ref[...] += jax.lax.dot_general(ohf, e, (((0,), (0,)), ((), ())),
                                      preferred_element_type=jnp.float32)
    for hd in range(HEADS):
        ow = ohf * e[:, hd:hd + 1]
        v_ref[:, hd * outf:(hd + 1) * outf] += jax.lax.dot_general(
            ow, hsrc[:, hd * outf:(hd + 1) * outf], (((0,), (0,)), ((), ())),
            preferred_element_type=jnp.float32)


def _glb_aggregate(h_src_nodes, W, a_src, er, bcol, outf):
    import functools
    fin = h_src_nodes.shape[1]
    hf = HEADS * outf
    aflat = a_src.reshape(1, hf)
    m = pl.pallas_call(
        functools.partial(_glb_max_k, outf=outf),
        grid=(NB,),
        in_specs=[pl.BlockSpec((BLK, fin), lambda i: (i, 0)),
                  pl.BlockSpec((fin, hf), lambda i: (0, 0)),
                  pl.BlockSpec((1, hf), lambda i: (0, 0)),
                  pl.BlockSpec((G, HEADS), lambda i: (0, 0)),
                  pl.BlockSpec((1, BLK, 1), lambda i: (i, 0, 0))],
        out_specs=pl.BlockSpec((HEADS, G), lambda i: (0, 0)),
        out_shape=jax.ShapeDtypeStruct((HEADS, G), jnp.float32),
        compiler_params=_ARB,
    )(h_src_nodes, W, aflat, er, bcol)
    msafeT = jnp.where(jnp.isfinite(m), m, 0.0).T
    u, v = pl.pallas_call(
        functools.partial(_glb_sum_k, outf=outf),
        grid=(NB,),
        in_specs=[pl.BlockSpec((BLK, fin), lambda i: (i, 0)),
                  pl.BlockSpec((fin, hf), lambda i: (0, 0)),
                  pl.BlockSpec((1, hf), lambda i: (0, 0)),
                  pl.BlockSpec((G, HEADS), lambda i: (0, 0)),
                  pl.BlockSpec((1, BLK, 1), lambda i: (i, 0, 0)),
                  pl.BlockSpec((G, HEADS), lambda i: (0, 0))],
        out_specs=[pl.BlockSpec((G, HEADS), lambda i: (0, 0)),
                   pl.BlockSpec((G, hf), lambda i: (0, 0))],
        out_shape=[jax.ShapeDtypeStruct((G, HEADS), jnp.float32),
                   jax.ShapeDtypeStruct((G, hf), jnp.float32)],
        compiler_params=_ARB,
    )(h_src_nodes, W, aflat, er, bcol, msafeT)
    rst = v.reshape(G, HEADS, outf) / (u[..., None] + 1e-9)
    return rst.reshape(G, hf)


def _upd_k(rst, x, resW, hgt, bcol, o_ref):
    oh = _onehot(bcol).astype(jnp.float32)
    out = rst[...] + jnp.dot(oh, hgt[...], preferred_element_type=jnp.float32)
    out += jnp.dot(x[...], resW[...], preferred_element_type=jnp.float32)
    o_ref[...] = _elu(out)


def _node_update(rst_edge, x, resW, hgt, bcol):
    fin = x.shape[1]
    hf = hgt.shape[1]
    return pl.pallas_call(
        _upd_k,
        grid=(NB,),
        in_specs=[pl.BlockSpec((BLK, hf), lambda i: (i, 0)),
                  pl.BlockSpec((BLK, fin), lambda i: (i, 0)),
                  pl.BlockSpec((fin, hf), lambda i: (0, 0)),
                  pl.BlockSpec((G, hf), lambda i: (0, 0)),
                  pl.BlockSpec((1, BLK, 1), lambda i: (i, 0, 0))],
        out_specs=pl.BlockSpec((BLK, hf), lambda i: (i, 0)),
        out_shape=jax.ShapeDtypeStruct((N_ATOM, hf), jnp.float32),
        compiler_params=_ARB,
    )(rst_edge, x, resW, hgt, bcol)


def _mm_k(x, W, o_ref):
    o_ref[...] = jnp.dot(x[...], W[...], preferred_element_type=jnp.float32)


def _dense_mm(x, W):
    fin, fout = W.shape
    return pl.pallas_call(
        _mm_k,
        grid=(NB,),
        in_specs=[pl.BlockSpec((BLK, fin), lambda i: (i, 0)),
                  pl.BlockSpec((fin, fout), lambda i: (0, 0))],
        out_specs=pl.BlockSpec((BLK, fout), lambda i: (i, 0)),
        out_shape=jax.ShapeDtypeStruct((x.shape[0], fout), jnp.float32),
        compiler_params=_ARB,
    )(x, W)


E_PAD = 102400


def _sc_gather(table, idx_p):
    D = table.shape[1]
    info = plsc.get_sparse_core_info()
    NW = info.num_cores * info.num_subcores
    b_per_w = E_PAD // NW
    C = 128
    n_chunks = b_per_w // C

    @functools.partial(
        pl.kernel,
        mesh=plsc.VectorSubcoreMesh(core_axis_name="c", subcore_axis_name="s"),
        out_type=jax.ShapeDtypeStruct((E_PAD, D), jnp.float32),
        scratch_types=[
            pltpu.VMEM((C,), jnp.int32),
            pltpu.VMEM((C, D), jnp.float32),
            pltpu.SemaphoreType.DMA,
        ],
    )
    def k(table_hbm, idx_hbm, out_hbm, idx_v, rows_v, sem):
        wid = jax.lax.axis_index("s") * info.num_cores + jax.lax.axis_index("c")
        base = wid * b_per_w

        def body(c, _):
            off = base + c * C
            pltpu.sync_copy(idx_hbm.at[pl.ds(off, C)], idx_v)
            pltpu.async_copy(table_hbm.at[idx_v], rows_v, sem).wait()
            pltpu.sync_copy(rows_v, out_hbm.at[pl.ds(off, C)])
            return 0

        jax.lax.fori_loop(0, n_chunks, body, 0)

    return k(table, idx_p)


def _gather_rows(table, idx):
    n = idx.shape[0]
    idx_p = jnp.concatenate(
        [idx.astype(jnp.int32), jnp.zeros((E_PAD - n,), jnp.int32)])
    return _sc_gather(table, idx_p)[:n]


def _fc_head_kernel(x_ref, w0, b0, w1, b1, w2, b2, w3, b3, o_ref):
    h = jnp.dot(x_ref[...], w0[...], preferred_element_type=jnp.float32) + b0[...]
    h = _elu(h)
    h = jnp.dot(h, w1[...], preferred_element_type=jnp.float32) + b1[...]
    h = _elu(h)
    h = jnp.dot(h, w2[...], preferred_element_type=jnp.float32) + b2[...]
    h = _elu(h)
    o_ref[...] = jnp.dot(h, w3[...], preferred_element_type=jnp.float32) + b3[...]


def _fc_head(x, fc):
    return pl.pallas_call(
        _fc_head_kernel,
        out_shape=jax.ShapeDtypeStruct((x.shape[0], 1), jnp.float32),
    )(x, fc[0]['W'], fc[0]['b'], fc[1]['W'], fc[1]['b'],
      fc[2]['W'], fc[2]['b'], fc[3]['W'], fc[3]['b'])


def kernel(feats_atom, feats_bond, feats_global, params, a2b_src, a2b_dst, atom_batch, bond_batch):
    atom_bcol = atom_batch.astype(jnp.float32).reshape(NB, BLK, 1)
    bond_bcol = bond_batch.astype(jnp.float32).reshape(NB, BLK, 1)
    bcols = {'atom': atom_bcol, 'bond': bond_bcol}
    nn_ = {'atom': N_ATOM, 'bond': N_BOND}
    h = {'atom': feats_atom, 'bond': feats_bond, 'global': feats_global}
    edges_ab = {'b2a': (a2b_dst, a2b_src), 'a2b': (a2b_src, a2b_dst)}

    for li, layer in enumerate(params['gat']):
        outf = GAT_HIDDEN[li]
        hf = HEADS * outf
        for m in NTYPES:
            pm = layer[m]
            x_m = h[m]
            if m == 'global':
                h_dst = (x_m @ pm['fc_master']).reshape(G, HEADS, outf)
                rst = jnp.zeros((G, hf), jnp.float32)
                for (s, e) in MECH[m]:
                    er = (h_dst * pm[e]['a_dst'][None]).sum(-1)
                    rst = rst + _glb_aggregate(h[s], pm[e]['W'], pm[e]['a_src'],
                                               er, bcols[s], outf)
                if li > 0:
                    rst = rst + x_m @ pm['res_fc']
                h[m] = jax.nn.elu(rst)
            else:
                Nm = nn_[m]
                (s, e) = MECH[m][0]
                src, dst = edges_ab[e]
                hsrc_flat = _dense_mm(h[s], pm[e]['W'])
                hdst_flat = _dense_mm(x_m, pm['fc_master'])
                el = (hsrc_flat.reshape(Nm, HEADS, outf)
                      * pm[e]['a_src'][None]).sum(-1)
                er = (hdst_flat.reshape(Nm, HEADS, outf)
                      * pm[e]['a_dst'][None]).sum(-1)
                logits = _lrelu(el[src] + er[dst])
                gmax = jnp.max(logits)
                gmax = jnp.where(jnp.isfinite(gmax), gmax, 0.0)
                ex = jnp.exp(logits - gmax)
                ssum = jax.ops.segment_sum(ex, dst, num_segments=Nm)
                alpha = ex / jnp.maximum(ssum[dst], 1e-30)
                rows = _gather_rows(hsrc_flat, src).reshape(-1, HEADS, outf)
                rst_edge = jax.ops.segment_sum(rows * alpha[..., None],
                                               dst, num_segments=Nm)
                rst_edge = rst_edge.reshape(Nm, hf)
                (sg, eg) = MECH[m][1]
                hgt = (h['global'] @ pm[eg]['W']) / (1.0 + 1e-9)
                resW = pm['res_fc'] if li > 0 else jnp.zeros((x_m.shape[1], hf),
                                                             jnp.float32)
                h[m] = _node_update(rst_edge, x_m, resW, hgt, bcols[m])

    def lstm_forward(layers, x, hs, cs):
        inp = x
        nh = []
        nc = []
        for i, p in enumerate(layers):
            z = inp @ p['Wi'] + hs[i] @ p['Wh'] + p['b']
            ig, fg, gg, og = jnp.split(z, 4, axis=-1)
            c = jax.nn.sigmoid(fg) * cs[i] + jax.nn.sigmoid(ig) * jnp.tanh(gg)
            hcur = jax.nn.sigmoid(og) * jnp.tanh(c)
            nh.append(hcur)
            nc.append(c)
            inp = hcur
        return inp, nh, nc

    def set2set(feat, bcol, layers):
        d = feat.shape[1]
        q_star = jnp.zeros((G, 2 * d), jnp.float32)
        hs = [jnp.zeros((G, d), jnp.float32) for _ in range(N_LSTM_LAYERS)]
        cs = [jnp.zeros((G, d), jnp.float32) for _ in range(N_LSTM_LAYERS)]
        for _ in range(N_ITERS):
            q, hs, cs = lstm_forward(layers, q_star, hs, cs)
            r = _s2s_attend(feat, q.T, bcol)
            q_star = jnp.concatenate([q, r], axis=-1)
        return q_star

    r_atom = set2set(h['atom'], atom_bcol, params['set2set']['atom'])
    r_bond = set2set(h['bond'], bond_bcol, params['set2set']['bond'])
    out = jnp.concatenate([r_atom, r_bond, h['global']], axis=-1)
    return _fc_head(out, params['fc'])
